# baseline (speedup 1.0000x reference)
import functools
import math

import jax
import jax.numpy as jnp
from jax import lax
from jax.experimental import pallas as pl
from jax.experimental.pallas import tpu as pltpu
from jax.experimental.pallas import tpu_sc as plsc

_N_NODES = 100000
_N_EDGES = 6400000
_NUM_TYPES = 64
_FACTOR = 1.0 / math.sqrt(64.0)

_NC = 2
_NS = 16
_NW = _NC * _NS

_CHUNK = 2048
_ROW = 128
_N_CHUNKS = _N_EDGES // _CHUNK
_DEPTH = 3

_ACC_PAD = 100352
_ACC_SLICE = _ACC_PAD // _NS


def _sc_body(ei_hbm, eng_hbm, spec_hbm, scal_hbm, out_hbm,
             cnb_v, eng_v, spec_v, zero_v, scal_v, acc,
             ld_sem, st_sem, tb_sem):
    cid = lax.axis_index("c")
    sid = lax.axis_index("s")
    wid = sid * _NC + cid

    n_chunks = (_N_CHUNKS - wid + _NW - 1) // _NW

    def _issue_loads(chunk, p):
        sl = pl.ds(chunk * _CHUNK, _CHUNK)
        pltpu.async_copy(ei_hbm.at[:, sl], cnb_v.at[p], ld_sem.at[p])
        pltpu.async_copy(eng_hbm.at[sl], eng_v.at[p], ld_sem.at[p])

    def _wait_loads(chunk, p):
        sl = pl.ds(chunk * _CHUNK, _CHUNK)
        pltpu.make_async_copy(ei_hbm.at[:, sl], cnb_v.at[p],
                              ld_sem.at[p]).wait()
        pltpu.make_async_copy(eng_hbm.at[sl], eng_v.at[p],
                              ld_sem.at[p]).wait()

    def _drain_scatters(p):
        for j in range(_CHUNK // _ROW):
            rsl = pl.ds(j * _ROW, _ROW)
            pltpu.make_async_copy(
                eng_v.at[p].at[rsl], acc.at[cnb_v.at[p].at[0].at[rsl]],
                st_sem.at[p]).wait()

    _issue_loads(wid, 0)
    _issue_loads(wid + _NW, 1)
    spec_copy = pltpu.async_copy(spec_hbm, spec_v, tb_sem)
    scal_copy = pltpu.async_copy(scal_hbm, scal_v, tb_sem)

    zeros16 = jnp.zeros((16,), jnp.float32)

    def _zero(i, carry):
        zero_v[pl.ds(i * 16, 16)] = zeros16
        return carry

    lax.fori_loop(0, _ACC_SLICE // 16, _zero, 0)
    pltpu.sync_copy(zero_v, acc.at[pl.ds(sid * _ACC_SLICE, _ACC_SLICE)])
    spec_copy.wait()
    scal_copy.wait()
    plsc.subcore_barrier()

    def _chunk(i, carry):
        p = lax.rem(i, _DEPTH)
        pn = lax.rem(i + 2, _DEPTH)
        chunk = wid + i * _NW

        @pl.when(i > 0)
        def _drain_prev():
            _drain_scatters(pn)

        @pl.when(i + 2 < n_chunks)
        def _prefetch():
            _issue_loads(chunk + 2 * _NW, pn)

        _wait_loads(chunk, p)

        for j in range(_CHUNK // _ROW):
            base = j * _ROW
            ngrp = _ROW // 16
            sls = [pl.ds(base + c * 16, 16) for c in range(ngrp)]
            cis = [cnb_v[p, 0, sl] for sl in sls]
            nis = [cnb_v[p, 1, sl] for sl in sls]
            cws = [plsc.load_gather(spec_v, [ci >> 2]) for ci in cis]
            nws = [plsc.load_gather(spec_v, [ni >> 2]) for ni in nis]
            spcs = [(cw >> ((ci & 3) << 3)) & 63
                    for cw, ci in zip(cws, cis)]
            spns = [(nw >> ((ni & 3) << 3)) & 63
                    for nw, ni in zip(nws, nis)]
            flats = [spc * _NUM_TYPES + spn
                     for spc, spn in zip(spcs, spns)]
            scs = [plsc.load_gather(scal_v, [flat]) for flat in flats]
            engs = [eng_v[p, sl] for sl in sls]
            for sl, e, sc in zip(sls, engs, scs):
                eng_v[p, sl] = e * sc
            rsl = pl.ds(base, _ROW)
            pltpu.async_copy(
                eng_v.at[p].at[rsl], acc.at[cnb_v.at[p].at[0].at[rsl]],
                st_sem.at[p], add=True)
        return carry

    lax.fori_loop(0, n_chunks, _chunk, 0)
    _drain_scatters(lax.rem(n_chunks - 1, _DEPTH))
    plsc.subcore_barrier()

    sl = pl.ds(sid * _ACC_SLICE, _ACC_SLICE)
    pltpu.sync_copy(acc.at[sl], out_hbm.at[cid, sl])


def _combine_body(p_ref, o_ref):
    o_ref[...] = (p_ref[0] + p_ref[1]) * _FACTOR


def kernel(edge_index, edge_energy, species, per_edge_scales):
    eng = edge_energy.reshape(_N_EDGES)
    sp4 = species.reshape(_N_NODES // 4, 4).astype(jnp.uint32)
    spec = (sp4[:, 0] | (sp4[:, 1] << 8) | (sp4[:, 2] << 16)
            | (sp4[:, 3] << 24)).astype(jnp.int32)
    scal = per_edge_scales.reshape(_NUM_TYPES * _NUM_TYPES)

    mesh = plsc.VectorSubcoreMesh(
        core_axis_name="c", subcore_axis_name="s",
        num_cores=_NC, num_subcores=_NS)

    partials = pl.kernel(
        _sc_body,
        out_type=jax.ShapeDtypeStruct((_NC, _ACC_PAD), jnp.float32),
        mesh=mesh,
        compiler_params=pltpu.CompilerParams(needs_layout_passes=False),
        scratch_types=[
            pltpu.VMEM((_DEPTH, 2, _CHUNK), jnp.int32),
            pltpu.VMEM((_DEPTH, _CHUNK), jnp.float32),
            pltpu.VMEM((_N_NODES // 4,), jnp.int32),
            pltpu.VMEM((_ACC_SLICE,), jnp.float32),
            pltpu.VMEM((_NUM_TYPES * _NUM_TYPES,), jnp.float32),
            pltpu.VMEM_SHARED((_ACC_PAD,), jnp.float32),
            pltpu.SemaphoreType.DMA((_DEPTH,)),
            pltpu.SemaphoreType.DMA((_DEPTH,)),
            pltpu.SemaphoreType.DMA,
        ],
    )(edge_index, eng, spec, scal)

    combined = pl.pallas_call(
        _combine_body,
        out_shape=jax.ShapeDtypeStruct((_ACC_PAD // 128, 128), jnp.float32),
    )(partials.reshape(_NC, _ACC_PAD // 128, 128))

    return combined.reshape(_ACC_PAD)[:_N_NODES, None]


# speedup vs baseline: 1.6236x; 1.6236x over previous
import functools
import math

import jax
import jax.numpy as jnp
from jax import lax
from jax.experimental import pallas as pl
from jax.experimental.pallas import tpu as pltpu
from jax.experimental.pallas import tpu_sc as plsc

_N_NODES = 100000
_N_EDGES = 6400000
_NUM_TYPES = 64
_FACTOR = 1.0 / math.sqrt(64.0)

_NC = 2
_NS = 16
_NW = _NC * _NS

_CHUNK = 1280
_DEPTH = 3
_ROW = 128
_N_CHUNKS = _N_EDGES // _CHUNK

_ACC_PAD = 100352
_ACC_SLICE = _ACC_PAD // _NS


def _sc_body(ei_hbm, eng_hbm, spec_hbm, scal_hbm, out_hbm,
             cnb_v, eng_v, spec_v, scal_v, zero_v, acc,
             ld_sem, st_sem, tb_sem):
    cid = lax.axis_index("c")
    sid = lax.axis_index("s")
    wid = sid * _NC + cid

    n_chunks = (_N_CHUNKS - wid + _NW - 1) // _NW

    def _issue_loads(chunk, p):
        sl = pl.ds(chunk * _CHUNK, _CHUNK)
        pltpu.async_copy(ei_hbm.at[:, sl], cnb_v.at[p], ld_sem.at[p])
        pltpu.async_copy(eng_hbm.at[sl], eng_v.at[p], ld_sem.at[p])

    def _wait_loads(chunk, p):
        sl = pl.ds(chunk * _CHUNK, _CHUNK)
        pltpu.make_async_copy(ei_hbm.at[:, sl], cnb_v.at[p],
                              ld_sem.at[p]).wait()
        pltpu.make_async_copy(eng_hbm.at[sl], eng_v.at[p],
                              ld_sem.at[p]).wait()

    def _drain_scatters(p):
        for j in range(_CHUNK // _ROW):
            rsl = pl.ds(j * _ROW, _ROW)
            pltpu.make_async_copy(
                eng_v.at[p].at[rsl], acc.at[cnb_v.at[p].at[0].at[rsl]],
                st_sem.at[p]).wait()

    _issue_loads(wid, 0)
    _issue_loads(wid + _NW, 1)
    spec_copy = pltpu.async_copy(spec_hbm, spec_v, tb_sem)
    scal_copy = pltpu.async_copy(scal_hbm, scal_v, tb_sem)

    zeros16 = jnp.zeros((16,), jnp.float32)

    def _zero(i, carry):
        zero_v[pl.ds(i * 16, 16)] = zeros16
        return carry

    lax.fori_loop(0, _ACC_SLICE // 16, _zero, 0)
    pltpu.sync_copy(zero_v, acc.at[pl.ds(sid * _ACC_SLICE, _ACC_SLICE)])
    spec_copy.wait()
    scal_copy.wait()
    plsc.subcore_barrier()

    def _chunk(i, carry):
        p = lax.rem(i, _DEPTH)
        pn = lax.rem(i + 2, _DEPTH)
        chunk = wid + i * _NW

        @pl.when(i > 0)
        def _drain_prev():
            _drain_scatters(pn)

        @pl.when(i + 2 < n_chunks)
        def _prefetch():
            _issue_loads(chunk + 2 * _NW, pn)

        _wait_loads(chunk, p)

        for j in range(_CHUNK // _ROW):
            base = j * _ROW
            ngrp = _ROW // 16
            sls = [pl.ds(base + c * 16, 16) for c in range(ngrp)]
            cis = [cnb_v[p, 0, sl] for sl in sls]
            nis = [cnb_v[p, 1, sl] for sl in sls]
            spcs = [plsc.load_gather(spec_v, [ci]) for ci in cis]
            spns = [plsc.load_gather(spec_v, [ni]) for ni in nis]
            flats = [spc * _NUM_TYPES + spn
                     for spc, spn in zip(spcs, spns)]
            scs = [plsc.load_gather(scal_v, [flat]) for flat in flats]
            engs = [eng_v[p, sl] for sl in sls]
            for sl, e, sc in zip(sls, engs, scs):
                eng_v[p, sl] = e * sc
            rsl = pl.ds(base, _ROW)
            pltpu.async_copy(
                eng_v.at[p].at[rsl], acc.at[cnb_v.at[p].at[0].at[rsl]],
                st_sem.at[p], add=True)
        return carry

    lax.fori_loop(0, n_chunks, _chunk, 0)
    _drain_scatters(lax.rem(n_chunks - 1, _DEPTH))
    plsc.subcore_barrier()

    sl = pl.ds(sid * _ACC_SLICE, _ACC_SLICE)
    pltpu.sync_copy(acc.at[sl], out_hbm.at[cid, sl])


def _combine_body(p_ref, o_ref):
    o_ref[...] = (p_ref[0] + p_ref[1]) * _FACTOR


def kernel(edge_index, edge_energy, species, per_edge_scales):
    eng = edge_energy.reshape(_N_EDGES)
    spec = species.reshape(_N_NODES)
    scal = per_edge_scales.reshape(_NUM_TYPES * _NUM_TYPES)

    mesh = plsc.VectorSubcoreMesh(
        core_axis_name="c", subcore_axis_name="s",
        num_cores=_NC, num_subcores=_NS)

    partials = pl.kernel(
        _sc_body,
        out_type=jax.ShapeDtypeStruct((_NC, _ACC_PAD), jnp.float32),
        mesh=mesh,
        compiler_params=pltpu.CompilerParams(needs_layout_passes=False),
        scratch_types=[
            pltpu.VMEM((_DEPTH, 2, _CHUNK), jnp.int32),
            pltpu.VMEM((_DEPTH, _CHUNK), jnp.float32),
            pltpu.VMEM((_N_NODES,), jnp.int32),
            pltpu.VMEM((_NUM_TYPES * _NUM_TYPES,), jnp.float32),
            pltpu.VMEM((_ACC_SLICE,), jnp.float32),
            pltpu.VMEM_SHARED((_ACC_PAD,), jnp.float32),
            pltpu.SemaphoreType.DMA((_DEPTH,)),
            pltpu.SemaphoreType.DMA((_DEPTH,)),
            pltpu.SemaphoreType.DMA,
        ],
    )(edge_index, eng, spec, scal)

    combined = pl.pallas_call(
        _combine_body,
        out_shape=jax.ShapeDtypeStruct((_ACC_PAD // 128, 128), jnp.float32),
    )(partials.reshape(_NC, _ACC_PAD // 128, 128))

    return combined.reshape(_ACC_PAD)[:_N_NODES, None]


# speedup vs baseline: 1.6244x; 1.0005x over previous
import functools
import math

import jax
import jax.numpy as jnp
from jax import lax
from jax.experimental import pallas as pl
from jax.experimental.pallas import tpu as pltpu
from jax.experimental.pallas import tpu_sc as plsc

_N_NODES = 100000
_N_EDGES = 6400000
_NUM_TYPES = 64
_FACTOR = 1.0 / math.sqrt(64.0)

_NC = 2
_NS = 16
_NW = _NC * _NS

_CHUNK = 1280
_DEPTH = 3
_ROW = 128
_N_CHUNKS = _N_EDGES // _CHUNK

_ACC_PAD = 100352
_ACC_SLICE = _ACC_PAD // _NS


def _sc_body(ei_hbm, eng_hbm, spec_hbm, scal_hbm, out_hbm,
             cnb_v, eng_v, spec_v, scal_v, zero_v, acc,
             ld_sem, st_sem, tb_sem):
    cid = lax.axis_index("c")
    sid = lax.axis_index("s")
    wid = sid * _NC + cid

    n_chunks = (_N_CHUNKS - wid + _NW - 1) // _NW

    def _issue_loads(chunk, p):
        sl = pl.ds(chunk * _CHUNK, _CHUNK)
        pltpu.async_copy(ei_hbm.at[:, sl], cnb_v.at[p], ld_sem.at[p])
        pltpu.async_copy(eng_hbm.at[sl], eng_v.at[p], ld_sem.at[p])

    def _wait_loads(chunk, p):
        sl = pl.ds(chunk * _CHUNK, _CHUNK)
        pltpu.make_async_copy(ei_hbm.at[:, sl], cnb_v.at[p],
                              ld_sem.at[p]).wait()
        pltpu.make_async_copy(eng_hbm.at[sl], eng_v.at[p],
                              ld_sem.at[p]).wait()

    def _drain_scatters(p):
        pltpu.make_async_copy(
            eng_hbm.at[pl.ds(0, _CHUNK)], eng_v.at[p],
            st_sem.at[p]).wait()

    _issue_loads(wid, 0)
    _issue_loads(wid + _NW, 1)
    spec_copy = pltpu.async_copy(spec_hbm, spec_v, tb_sem)
    scal_copy = pltpu.async_copy(scal_hbm, scal_v, tb_sem)

    zeros16 = jnp.zeros((16,), jnp.float32)

    def _zero(i, carry):
        zero_v[pl.ds(i * 16, 16)] = zeros16
        return carry

    lax.fori_loop(0, _ACC_SLICE // 16, _zero, 0)
    pltpu.sync_copy(zero_v, acc.at[pl.ds(sid * _ACC_SLICE, _ACC_SLICE)])
    spec_copy.wait()
    scal_copy.wait()
    plsc.subcore_barrier()

    def _chunk(i, carry):
        p = lax.rem(i, _DEPTH)
        pn = lax.rem(i + 2, _DEPTH)
        chunk = wid + i * _NW

        @pl.when(i > 0)
        def _drain_prev():
            _drain_scatters(pn)

        @pl.when(i + 2 < n_chunks)
        def _prefetch():
            _issue_loads(chunk + 2 * _NW, pn)

        _wait_loads(chunk, p)

        for j in range(_CHUNK // _ROW):
            base = j * _ROW
            ngrp = _ROW // 16
            sls = [pl.ds(base + c * 16, 16) for c in range(ngrp)]
            cis = [cnb_v[p, 0, sl] for sl in sls]
            nis = [cnb_v[p, 1, sl] for sl in sls]
            spcs = [plsc.load_gather(spec_v, [ci]) for ci in cis]
            spns = [plsc.load_gather(spec_v, [ni]) for ni in nis]
            flats = [spc * _NUM_TYPES + spn
                     for spc, spn in zip(spcs, spns)]
            scs = [plsc.load_gather(scal_v, [flat]) for flat in flats]
            engs = [eng_v[p, sl] for sl in sls]
            for sl, e, sc in zip(sls, engs, scs):
                eng_v[p, sl] = e * sc
            rsl = pl.ds(base, _ROW)
            pltpu.async_copy(
                eng_v.at[p].at[rsl], acc.at[cnb_v.at[p].at[0].at[rsl]],
                st_sem.at[p], add=True)
        return carry

    lax.fori_loop(0, n_chunks, _chunk, 0)
    _drain_scatters(lax.rem(n_chunks - 1, _DEPTH))
    plsc.subcore_barrier()

    sl = pl.ds(sid * _ACC_SLICE, _ACC_SLICE)
    pltpu.sync_copy(acc.at[sl], out_hbm.at[cid, sl])


def _combine_body(p_ref, o_ref):
    o_ref[...] = (p_ref[0] + p_ref[1]) * _FACTOR


def kernel(edge_index, edge_energy, species, per_edge_scales):
    eng = edge_energy.reshape(_N_EDGES)
    spec = species.reshape(_N_NODES)
    scal = per_edge_scales.reshape(_NUM_TYPES * _NUM_TYPES)

    mesh = plsc.VectorSubcoreMesh(
        core_axis_name="c", subcore_axis_name="s",
        num_cores=_NC, num_subcores=_NS)

    partials = pl.kernel(
        _sc_body,
        out_type=jax.ShapeDtypeStruct((_NC, _ACC_PAD), jnp.float32),
        mesh=mesh,
        compiler_params=pltpu.CompilerParams(needs_layout_passes=False),
        scratch_types=[
            pltpu.VMEM((_DEPTH, 2, _CHUNK), jnp.int32),
            pltpu.VMEM((_DEPTH, _CHUNK), jnp.float32),
            pltpu.VMEM((_N_NODES,), jnp.int32),
            pltpu.VMEM((_NUM_TYPES * _NUM_TYPES,), jnp.float32),
            pltpu.VMEM((_ACC_SLICE,), jnp.float32),
            pltpu.VMEM_SHARED((_ACC_PAD,), jnp.float32),
            pltpu.SemaphoreType.DMA((_DEPTH,)),
            pltpu.SemaphoreType.DMA((_DEPTH,)),
            pltpu.SemaphoreType.DMA,
        ],
    )(edge_index, eng, spec, scal)

    combined = pl.pallas_call(
        _combine_body,
        out_shape=jax.ShapeDtypeStruct((_ACC_PAD // 128, 128), jnp.float32),
    )(partials.reshape(_NC, _ACC_PAD // 128, 128))

    return combined.reshape(_ACC_PAD)[:_N_NODES, None]
